# baseline (speedup 1.0000x reference)
import functools

import jax
import jax.numpy as jnp
from jax import lax
from jax.experimental import pallas as pl
from jax.experimental.pallas import tpu as pltpu
from jax.experimental.pallas import tpu_sc as plsc

_K = 3
_MASK_PENALTY = float(1e10)
_BIG_D = float(1e30)


_QB = 1024
_KC = 1024


def _lex_insert(bd, bi, m, i):
    out_d, out_i = [], []
    for t in range(_K):
        out_d.append(bd[t])
        out_i.append(bi[t])
    def lt(cd, ci, td, ti):
        return (cd < td) | ((cd == td) & (ci < ti))
    c0 = lt(m, i, bd[0], bi[0])
    c1 = lt(m, i, bd[1], bi[1])
    c2 = lt(m, i, bd[2], bi[2])
    n2d = jnp.where(c2, jnp.where(c1, bd[1], m), bd[2])
    n2i = jnp.where(c2, jnp.where(c1, bi[1], i), bi[2])
    n1d = jnp.where(c1, jnp.where(c0, bd[0], m), bd[1])
    n1i = jnp.where(c1, jnp.where(c0, bi[0], i), bi[1])
    n0d = jnp.where(c0, m, bd[0])
    n0i = jnp.where(c0, i, bi[0])
    return [n0d, n1d, n2d], [n0i, n1i, n2i]


def _knn_body(nk, nkc, qpos_ref, qb_ref, kpos_ref, kb_ref, idx_ref, wn_ref,
              cd_ref, ci_ref):
    qb = qb_ref[:, :]
    qbmin = jnp.min(qb)
    qbmax = jnp.max(qb)
    ncand = _K * nkc + _K
    cd_ref[:, :] = jnp.full((_QB, ncand), _BIG_D, jnp.float32)
    ci_ref[:, :] = jnp.full((_QB, ncand), float(nk), jnp.float32)

    for k in range(_K):
        c = _K * nkc + k
        cd_ref[:, c:c + 1] = jnp.where(qb == kb_ref[:, k:k + 1], _BIG_D,
                                       _MASK_PENALTY)
        ci_ref[:, c:c + 1] = jnp.full((_QB, 1), float(k), jnp.float32)

    qx = qpos_ref[:, 0:1]
    qy = qpos_ref[:, 1:2]
    qz = qpos_ref[:, 2:3]
    iota0 = lax.broadcasted_iota(jnp.int32, (_QB, _KC), 1).astype(jnp.float32)

    for j in range(nkc):
        def process(j=j):
            sl = pl.ds(j * _KC, _KC)
            kb_c = kb_ref[:, sl]
            d = (qx - kpos_ref[0:1, sl]) ** 2
            d = d + (qy - kpos_ref[1:2, sl]) ** 2
            d = d + (qz - kpos_ref[2:3, sl]) ** 2
            d = d + jnp.where(qb != kb_c, _MASK_PENALTY, jnp.float32(0.0))
            iota = iota0 + jnp.float32(j * _KC)
            for t in range(_K):
                m = jnp.min(d, axis=1, keepdims=True)
                i = jnp.min(jnp.where(d == m, iota, jnp.float32(nk)),
                            axis=1, keepdims=True)
                cd_ref[:, j * _K + t:j * _K + t + 1] = m
                ci_ref[:, j * _K + t:j * _K + t + 1] = i
                if t < _K - 1:
                    d = jnp.where(iota == i, _BIG_D, d)

        kb_c = kb_ref[:, pl.ds(j * _KC, _KC)]
        kcmin = jnp.min(kb_c)
        kcmax = jnp.max(kb_c)
        pl.when((kcmin <= qbmax) & (kcmax >= qbmin))(process)

    dc = cd_ref[:, :]
    ic = ci_ref[:, :]
    sel_d, sel_i = [], []
    for t in range(_K):
        m = jnp.min(dc, axis=1, keepdims=True)
        i = jnp.min(jnp.where(dc == m, ic, jnp.float32(nk)),
                    axis=1, keepdims=True)
        sel_d.append(m)
        sel_i.append(i)
        if t < _K - 1:
            dc = jnp.where((dc == m) & (ic == i), _BIG_D, dc)

    w = [jnp.float32(1.0) / jnp.maximum(sel_d[t], jnp.float32(1e-16))
         for t in range(_K)]
    den = w[0] + w[1] + w[2]
    for t in range(_K):
        idx_ref[:, t:t + 1] = sel_i[t].astype(jnp.int32)
        wn_ref[:, t * 16:(t + 1) * 16] = jnp.broadcast_to(w[t] / den,
                                                          (_QB, 16))
    idx_ref[:, _K:_K + 1] = jnp.zeros((_QB, 1), jnp.int32)


def _knn_topk(pos, batch_f, pos_skip, batch_skip_f):
    n2 = pos_skip.shape[0]
    nk = pos.shape[0]
    nkc = nk // _KC
    grid = n2 // _QB
    kpos_t = pos.T
    kb = batch_f.reshape(1, nk)
    qb = batch_skip_f.reshape(n2, 1)
    idx_q, wn_q = pl.pallas_call(
        functools.partial(_knn_body, nk, nkc),
        grid=(grid,),
        in_specs=[
            pl.BlockSpec((_QB, 3), lambda i: (i, 0)),
            pl.BlockSpec((_QB, 1), lambda i: (i, 0)),
            pl.BlockSpec((3, nk), lambda i: (0, 0)),
            pl.BlockSpec((1, nk), lambda i: (0, 0)),
        ],
        out_specs=[
            pl.BlockSpec((_QB, _K + 1), lambda i: (i, 0)),
            pl.BlockSpec((_QB, _K * 16), lambda i: (i, 0)),
        ],
        out_shape=[
            jax.ShapeDtypeStruct((n2, _K + 1), jnp.int32),
            jax.ShapeDtypeStruct((n2, _K * 16), jnp.float32),
        ],
        scratch_shapes=[
            pltpu.VMEM((_QB, _K * nkc + _K), jnp.float32),
            pltpu.VMEM((_QB, _K * nkc + _K), jnp.float32),
        ],
    )(pos_skip, qb, kpos_t, kb)
    return idx_q, wn_q


_NC = 2
_NS = 16
_NW = _NC * _NS
_CH = 16


def _sc_gather_body(n2, d_in, x_hbm, i0_hbm, i1_hbm, i2_hbm, wrep_hbm, y_hbm,
                    i0_v, i1_v, i2_v,
                    r0a, r1a, r2a, wca, ya,
                    r0b, r1b, r2b, wcb, yb,
                    gsem0, gsem1, ssem0, ssem1):
    qpw = n2 // _NW
    nch = qpw // _CH
    wid = lax.axis_index("s") * _NC + lax.axis_index("c")
    wbase = wid * qpw

    pltpu.sync_copy(i0_hbm.at[pl.ds(wid * nch, nch)], i0_v)
    pltpu.sync_copy(i1_hbm.at[pl.ds(wid * nch, nch)], i1_v)
    pltpu.sync_copy(i2_hbm.at[pl.ds(wid * nch, nch)], i2_v)

    bufs = ((r0a, r1a, r2a, wca, ya, gsem0, ssem0),
            (r0b, r1b, r2b, wcb, yb, gsem1, ssem1))

    def fire(b, t):
        r0, r1, r2, wc, _, gsem, _s = bufs[b]
        pltpu.async_copy(x_hbm.at[i0_v.at[t]], r0, gsem)
        pltpu.async_copy(x_hbm.at[i1_v.at[t]], r1, gsem)
        pltpu.async_copy(x_hbm.at[i2_v.at[t]], r2, gsem)
        pltpu.async_copy(wrep_hbm.at[pl.ds(wbase + t * _CH, _CH)], wc, gsem)

    def drain_gathers(b):
        r0, r1, r2, wc, _, gsem, _s = bufs[b]
        pltpu.make_async_copy(x_hbm.at[i0_v.at[0]], r0, gsem).wait()
        pltpu.make_async_copy(x_hbm.at[i1_v.at[0]], r1, gsem).wait()
        pltpu.make_async_copy(x_hbm.at[i2_v.at[0]], r2, gsem).wait()
        pltpu.make_async_copy(wrep_hbm.at[pl.ds(wbase, _CH)], wc, gsem).wait()

    def wait_store(b):
        y = bufs[b][4]
        ssem = bufs[b][6]
        pltpu.make_async_copy(y, y_hbm.at[pl.ds(wbase, _CH)], ssem).wait()

    fire(0, 0)
    fire(1, 1)

    @pl.loop(0, nch, step=2)
    def _steps(t):
        for b in range(2):
            tt = t + b
            r0, r1, r2, wc, y, gsem, ssem = bufs[b]

            @pl.when(tt >= 2)
            def _():
                wait_store(b)

            drain_gathers(b)
            for q in range(_CH):
                w0 = wc[q, pl.ds(0, 16)]
                w1 = wc[q, pl.ds(16, 16)]
                w2 = wc[q, pl.ds(32, 16)]
                for f in range(d_in // 16):
                    fs = pl.ds(f * 16, 16)
                    y[q, fs] = (w0 * r0[q, fs] + w1 * r1[q, fs]
                                + w2 * r2[q, fs])
            pltpu.async_copy(y, y_hbm.at[pl.ds(wbase + tt * _CH, _CH)], ssem)

            @pl.when(tt + 2 < nch)
            def _():
                fire(b, tt + 2)

    wait_store(0)
    wait_store(1)


def _sc_gather_combine(x, idx_q, wn_rep):
    n2 = idx_q.shape[0]
    d_in = x.shape[1]
    qpw = n2 // _NW
    nch = qpw // _CH
    nch_total = n2 // _CH
    idx3 = idx_q[:, :_K].T.reshape(_K, nch_total, _CH)
    mesh = plsc.VectorSubcoreMesh(core_axis_name="c", subcore_axis_name="s")
    y = pl.kernel(
        functools.partial(_sc_gather_body, n2, d_in),
        out_type=jax.ShapeDtypeStruct((n2, d_in), jnp.float32),
        mesh=mesh,
        scratch_types=[
            pltpu.VMEM((nch, _CH), jnp.int32),
            pltpu.VMEM((nch, _CH), jnp.int32),
            pltpu.VMEM((nch, _CH), jnp.int32),
            pltpu.VMEM((_CH, d_in), jnp.float32),
            pltpu.VMEM((_CH, d_in), jnp.float32),
            pltpu.VMEM((_CH, d_in), jnp.float32),
            pltpu.VMEM((_CH, _K * 16), jnp.float32),
            pltpu.VMEM((_CH, d_in), jnp.float32),
            pltpu.VMEM((_CH, d_in), jnp.float32),
            pltpu.VMEM((_CH, d_in), jnp.float32),
            pltpu.VMEM((_CH, d_in), jnp.float32),
            pltpu.VMEM((_CH, _K * 16), jnp.float32),
            pltpu.VMEM((_CH, d_in), jnp.float32),
            pltpu.SemaphoreType.DMA,
            pltpu.SemaphoreType.DMA,
            pltpu.SemaphoreType.DMA,
            pltpu.SemaphoreType.DMA,
        ],
    )(x, idx3[0], idx3[1], idx3[2], wn_rep)
    return y


_RB = 512


def _mlp_body(y_ref, xs_ref, w1_ref, b1_ref, w2_ref, b2_ref, out_ref):
    h = jnp.concatenate([y_ref[:, :], xs_ref[:, :]], axis=1)
    z1 = jnp.dot(h, w1_ref[:, :], preferred_element_type=jnp.float32,
                 precision=lax.Precision.HIGHEST) + b1_ref[:, :]
    a1 = jnp.maximum(z1, 0.0)
    z2 = jnp.dot(a1, w2_ref[:, :], preferred_element_type=jnp.float32,
                 precision=lax.Precision.HIGHEST) + b2_ref[:, :]
    out_ref[:, :] = jnp.maximum(z2, 0.0)


def _mlp(y, x_skip, W1, b1, W2, b2):
    n2, d_in = y.shape
    d_skip = x_skip.shape[1]
    d_mid = W1.shape[1]
    d_out = W2.shape[1]
    grid = n2 // _RB
    return pl.pallas_call(
        _mlp_body,
        grid=(grid,),
        in_specs=[
            pl.BlockSpec((_RB, d_in), lambda i: (i, 0)),
            pl.BlockSpec((_RB, d_skip), lambda i: (i, 0)),
            pl.BlockSpec((d_in + d_skip, d_mid), lambda i: (0, 0)),
            pl.BlockSpec((1, d_mid), lambda i: (0, 0)),
            pl.BlockSpec((d_mid, d_out), lambda i: (0, 0)),
            pl.BlockSpec((1, d_out), lambda i: (0, 0)),
        ],
        out_specs=pl.BlockSpec((_RB, d_out), lambda i: (i, 0)),
        out_shape=jax.ShapeDtypeStruct((n2, d_out), jnp.float32),
    )(y, x_skip, W1, b1.reshape(1, d_mid), W2, b2.reshape(1, d_out))


def kernel(x, pos, batch, x_skip, pos_skip, batch_skip, W1, b1, W2, b2):
    batch_f = batch.astype(jnp.float32)
    batch_skip_f = batch_skip.astype(jnp.float32)
    n2 = pos_skip.shape[0]
    ns = 4
    s = n2 // ns
    tops = [_knn_topk(pos, batch_f, pos_skip[i * s:(i + 1) * s],
                      batch_skip_f[i * s:(i + 1) * s]) for i in range(ns)]
    ys = [_sc_gather_combine(x, idx_i, wn_i) for idx_i, wn_i in tops]
    hs = [_mlp(y_i, x_skip[i * s:(i + 1) * s], W1, b1, W2, b2)
          for i, y_i in enumerate(ys)]
    h = jnp.concatenate(hs, axis=0)
    return (h, pos_skip, batch_skip)


# speedup vs baseline: 1.1453x; 1.1453x over previous
import functools

import jax
import jax.numpy as jnp
from jax import lax
from jax.experimental import pallas as pl
from jax.experimental.pallas import tpu as pltpu
from jax.experimental.pallas import tpu_sc as plsc

_K = 3
_MASK_PENALTY = float(1e10)
_BIG_D = float(1e30)


_QB = 1024
_KC = 512


def _lex_insert(bd, bi, m, i):
    out_d, out_i = [], []
    for t in range(_K):
        out_d.append(bd[t])
        out_i.append(bi[t])
    def lt(cd, ci, td, ti):
        return (cd < td) | ((cd == td) & (ci < ti))
    c0 = lt(m, i, bd[0], bi[0])
    c1 = lt(m, i, bd[1], bi[1])
    c2 = lt(m, i, bd[2], bi[2])
    n2d = jnp.where(c2, jnp.where(c1, bd[1], m), bd[2])
    n2i = jnp.where(c2, jnp.where(c1, bi[1], i), bi[2])
    n1d = jnp.where(c1, jnp.where(c0, bd[0], m), bd[1])
    n1i = jnp.where(c1, jnp.where(c0, bi[0], i), bi[1])
    n0d = jnp.where(c0, m, bd[0])
    n0i = jnp.where(c0, i, bi[0])
    return [n0d, n1d, n2d], [n0i, n1i, n2i]


def _knn_body(nk, nkc, qpos_ref, qb_ref, kpos_ref, kb_ref, idx_ref, wn_ref,
              cd_ref, ci_ref):
    qb = qb_ref[:, :]
    qbmin = jnp.min(qb)
    qbmax = jnp.max(qb)
    ncand = _K * nkc + _K
    cd_ref[:, :] = jnp.full((_QB, ncand), _BIG_D, jnp.float32)
    ci_ref[:, :] = jnp.full((_QB, ncand), float(nk), jnp.float32)

    for k in range(_K):
        c = _K * nkc + k
        cd_ref[:, c:c + 1] = jnp.where(qb == kb_ref[:, k:k + 1], _BIG_D,
                                       _MASK_PENALTY)
        ci_ref[:, c:c + 1] = jnp.full((_QB, 1), float(k), jnp.float32)

    qx = qpos_ref[:, 0:1]
    qy = qpos_ref[:, 1:2]
    qz = qpos_ref[:, 2:3]
    iota0 = lax.broadcasted_iota(jnp.int32, (_QB, _KC), 1).astype(jnp.float32)

    for j in range(nkc):
        def process(j=j):
            sl = pl.ds(j * _KC, _KC)
            kb_c = kb_ref[:, sl]
            d = (qx - kpos_ref[0:1, sl]) ** 2
            d = d + (qy - kpos_ref[1:2, sl]) ** 2
            d = d + (qz - kpos_ref[2:3, sl]) ** 2
            d = d + jnp.where(qb != kb_c, _MASK_PENALTY, jnp.float32(0.0))
            iota = iota0 + jnp.float32(j * _KC)
            for t in range(_K):
                m = jnp.min(d, axis=1, keepdims=True)
                i = jnp.min(jnp.where(d == m, iota, jnp.float32(nk)),
                            axis=1, keepdims=True)
                cd_ref[:, j * _K + t:j * _K + t + 1] = m
                ci_ref[:, j * _K + t:j * _K + t + 1] = i
                if t < _K - 1:
                    d = jnp.where(iota == i, _BIG_D, d)

        kb_c = kb_ref[:, pl.ds(j * _KC, _KC)]
        kcmin = jnp.min(kb_c)
        kcmax = jnp.max(kb_c)
        pl.when((kcmin <= qbmax) & (kcmax >= qbmin))(process)

    dc = cd_ref[:, :]
    ic = ci_ref[:, :]
    sel_d, sel_i = [], []
    for t in range(_K):
        m = jnp.min(dc, axis=1, keepdims=True)
        i = jnp.min(jnp.where(dc == m, ic, jnp.float32(nk)),
                    axis=1, keepdims=True)
        sel_d.append(m)
        sel_i.append(i)
        if t < _K - 1:
            dc = jnp.where((dc == m) & (ic == i), _BIG_D, dc)

    w = [jnp.float32(1.0) / jnp.maximum(sel_d[t], jnp.float32(1e-16))
         for t in range(_K)]
    den = w[0] + w[1] + w[2]
    for t in range(_K):
        idx_ref[:, t:t + 1] = sel_i[t].astype(jnp.int32)
        wn_ref[:, t * 16:(t + 1) * 16] = jnp.broadcast_to(w[t] / den,
                                                          (_QB, 16))
    idx_ref[:, _K:_K + 1] = jnp.zeros((_QB, 1), jnp.int32)


def _knn_topk(pos, batch_f, pos_skip, batch_skip_f):
    n2 = pos_skip.shape[0]
    nk = pos.shape[0]
    nkc = nk // _KC
    grid = n2 // _QB
    kpos_t = pos.T
    kb = batch_f.reshape(1, nk)
    qb = batch_skip_f.reshape(n2, 1)
    idx_q, wn_q = pl.pallas_call(
        functools.partial(_knn_body, nk, nkc),
        grid=(grid,),
        in_specs=[
            pl.BlockSpec((_QB, 3), lambda i: (i, 0)),
            pl.BlockSpec((_QB, 1), lambda i: (i, 0)),
            pl.BlockSpec((3, nk), lambda i: (0, 0)),
            pl.BlockSpec((1, nk), lambda i: (0, 0)),
        ],
        out_specs=[
            pl.BlockSpec((_QB, _K + 1), lambda i: (i, 0)),
            pl.BlockSpec((_QB, _K * 16), lambda i: (i, 0)),
        ],
        out_shape=[
            jax.ShapeDtypeStruct((n2, _K + 1), jnp.int32),
            jax.ShapeDtypeStruct((n2, _K * 16), jnp.float32),
        ],
        scratch_shapes=[
            pltpu.VMEM((_QB, _K * nkc + _K), jnp.float32),
            pltpu.VMEM((_QB, _K * nkc + _K), jnp.float32),
        ],
    )(pos_skip, qb, kpos_t, kb)
    return idx_q, wn_q


_NC = 2
_NS = 16
_NW = _NC * _NS
_CH = 16


def _sc_gather_body(n2, d_in, x_hbm, i0_hbm, i1_hbm, i2_hbm, wrep_hbm, y_hbm,
                    i0_v, i1_v, i2_v,
                    r0a, r1a, r2a, wca, ya,
                    r0b, r1b, r2b, wcb, yb,
                    gsem0, gsem1, ssem0, ssem1):
    qpw = n2 // _NW
    nch = qpw // _CH
    wid = lax.axis_index("s") * _NC + lax.axis_index("c")
    wbase = wid * qpw

    pltpu.sync_copy(i0_hbm.at[pl.ds(wid * nch, nch)], i0_v)
    pltpu.sync_copy(i1_hbm.at[pl.ds(wid * nch, nch)], i1_v)
    pltpu.sync_copy(i2_hbm.at[pl.ds(wid * nch, nch)], i2_v)

    bufs = ((r0a, r1a, r2a, wca, ya, gsem0, ssem0),
            (r0b, r1b, r2b, wcb, yb, gsem1, ssem1))

    def fire(b, t):
        r0, r1, r2, wc, _, gsem, _s = bufs[b]
        pltpu.async_copy(x_hbm.at[i0_v.at[t]], r0, gsem)
        pltpu.async_copy(x_hbm.at[i1_v.at[t]], r1, gsem)
        pltpu.async_copy(x_hbm.at[i2_v.at[t]], r2, gsem)
        pltpu.async_copy(wrep_hbm.at[pl.ds(wbase + t * _CH, _CH)], wc, gsem)

    def drain_gathers(b):
        r0, r1, r2, wc, _, gsem, _s = bufs[b]
        pltpu.make_async_copy(x_hbm.at[i0_v.at[0]], r0, gsem).wait()
        pltpu.make_async_copy(x_hbm.at[i1_v.at[0]], r1, gsem).wait()
        pltpu.make_async_copy(x_hbm.at[i2_v.at[0]], r2, gsem).wait()
        pltpu.make_async_copy(wrep_hbm.at[pl.ds(wbase, _CH)], wc, gsem).wait()

    def wait_store(b):
        y = bufs[b][4]
        ssem = bufs[b][6]
        pltpu.make_async_copy(y, y_hbm.at[pl.ds(wbase, _CH)], ssem).wait()

    fire(0, 0)
    fire(1, 1)

    @pl.loop(0, nch, step=2)
    def _steps(t):
        for b in range(2):
            tt = t + b
            r0, r1, r2, wc, y, gsem, ssem = bufs[b]

            @pl.when(tt >= 2)
            def _():
                wait_store(b)

            drain_gathers(b)
            for q in range(_CH):
                w0 = wc[q, pl.ds(0, 16)]
                w1 = wc[q, pl.ds(16, 16)]
                w2 = wc[q, pl.ds(32, 16)]
                for f in range(d_in // 16):
                    fs = pl.ds(f * 16, 16)
                    y[q, fs] = (w0 * r0[q, fs] + w1 * r1[q, fs]
                                + w2 * r2[q, fs])
            pltpu.async_copy(y, y_hbm.at[pl.ds(wbase + tt * _CH, _CH)], ssem)

            @pl.when(tt + 2 < nch)
            def _():
                fire(b, tt + 2)

    wait_store(0)
    wait_store(1)


def _sc_gather_combine(x, idx_q, wn_rep):
    n2 = idx_q.shape[0]
    d_in = x.shape[1]
    qpw = n2 // _NW
    nch = qpw // _CH
    nch_total = n2 // _CH
    idx3 = idx_q[:, :_K].T.reshape(_K, nch_total, _CH)
    mesh = plsc.VectorSubcoreMesh(core_axis_name="c", subcore_axis_name="s")
    y = pl.kernel(
        functools.partial(_sc_gather_body, n2, d_in),
        out_type=jax.ShapeDtypeStruct((n2, d_in), jnp.float32),
        mesh=mesh,
        scratch_types=[
            pltpu.VMEM((nch, _CH), jnp.int32),
            pltpu.VMEM((nch, _CH), jnp.int32),
            pltpu.VMEM((nch, _CH), jnp.int32),
            pltpu.VMEM((_CH, d_in), jnp.float32),
            pltpu.VMEM((_CH, d_in), jnp.float32),
            pltpu.VMEM((_CH, d_in), jnp.float32),
            pltpu.VMEM((_CH, _K * 16), jnp.float32),
            pltpu.VMEM((_CH, d_in), jnp.float32),
            pltpu.VMEM((_CH, d_in), jnp.float32),
            pltpu.VMEM((_CH, d_in), jnp.float32),
            pltpu.VMEM((_CH, d_in), jnp.float32),
            pltpu.VMEM((_CH, _K * 16), jnp.float32),
            pltpu.VMEM((_CH, d_in), jnp.float32),
            pltpu.SemaphoreType.DMA,
            pltpu.SemaphoreType.DMA,
            pltpu.SemaphoreType.DMA,
            pltpu.SemaphoreType.DMA,
        ],
    )(x, idx3[0], idx3[1], idx3[2], wn_rep)
    return y


_RB = 512


def _mlp_body(y_ref, xs_ref, w1_ref, b1_ref, w2_ref, b2_ref, out_ref):
    h = jnp.concatenate([y_ref[:, :], xs_ref[:, :]], axis=1)
    z1 = jnp.dot(h, w1_ref[:, :], preferred_element_type=jnp.float32,
                 precision=lax.Precision.HIGHEST) + b1_ref[:, :]
    a1 = jnp.maximum(z1, 0.0)
    z2 = jnp.dot(a1, w2_ref[:, :], preferred_element_type=jnp.float32,
                 precision=lax.Precision.HIGHEST) + b2_ref[:, :]
    out_ref[:, :] = jnp.maximum(z2, 0.0)


def _mlp(y, x_skip, W1, b1, W2, b2):
    n2, d_in = y.shape
    d_skip = x_skip.shape[1]
    d_mid = W1.shape[1]
    d_out = W2.shape[1]
    grid = n2 // _RB
    return pl.pallas_call(
        _mlp_body,
        grid=(grid,),
        in_specs=[
            pl.BlockSpec((_RB, d_in), lambda i: (i, 0)),
            pl.BlockSpec((_RB, d_skip), lambda i: (i, 0)),
            pl.BlockSpec((d_in + d_skip, d_mid), lambda i: (0, 0)),
            pl.BlockSpec((1, d_mid), lambda i: (0, 0)),
            pl.BlockSpec((d_mid, d_out), lambda i: (0, 0)),
            pl.BlockSpec((1, d_out), lambda i: (0, 0)),
        ],
        out_specs=pl.BlockSpec((_RB, d_out), lambda i: (i, 0)),
        out_shape=jax.ShapeDtypeStruct((n2, d_out), jnp.float32),
    )(y, x_skip, W1, b1.reshape(1, d_mid), W2, b2.reshape(1, d_out))


def kernel(x, pos, batch, x_skip, pos_skip, batch_skip, W1, b1, W2, b2):
    batch_f = batch.astype(jnp.float32)
    batch_skip_f = batch_skip.astype(jnp.float32)
    n2 = pos_skip.shape[0]
    ns = 2
    s = n2 // ns
    tops = [_knn_topk(pos, batch_f, pos_skip[i * s:(i + 1) * s],
                      batch_skip_f[i * s:(i + 1) * s]) for i in range(ns)]
    ys = [_sc_gather_combine(x, idx_i, wn_i) for idx_i, wn_i in tops]
    hs = [_mlp(y_i, x_skip[i * s:(i + 1) * s], W1, b1, W2, b2)
          for i, y_i in enumerate(ys)]
    h = jnp.concatenate(hs, axis=0)
    return (h, pos_skip, batch_skip)
